# baseline (speedup 1.0000x reference)
import functools

import jax
import jax.numpy as jnp
from jax import lax
from jax.experimental import pallas as pl
from jax.experimental.pallas import tpu as pltpu, tpu_sc as plsc

_NC = 2
_NS = 16
_NW = _NC * _NS
_L = 16

_CH = 32


def _sc_body(x_hbm, pe_hbm, out_hbm, x_v, pe_v):
    batch, seq_len, d = x_hbm.shape
    rows_w = seq_len // _NW
    nch = rows_w // _CH
    wid = lax.axis_index("s") * _NC + lax.axis_index("c")
    base = wid * rows_w

    def chunk_body(ci, _):
        s0 = base + ci * _CH
        pltpu.sync_copy(pe_hbm.at[pl.ds(s0, _CH)], pe_v)
        for b in range(batch):
            pltpu.sync_copy(x_hbm.at[b, pl.ds(s0, _CH)], x_v)

            def row_body(r, _):
                for j in range(d // _L):
                    sl = pl.ds(j * _L, _L)
                    x_v[r, sl] = x_v[r, sl] + pe_v[r, sl]
                return 0

            lax.fori_loop(0, _CH, row_body, 0)
            pltpu.sync_copy(x_v, out_hbm.at[b, pl.ds(s0, _CH)])
        return 0

    lax.fori_loop(0, nch, chunk_body, 0)


def kernel(x, pe_table):
    batch, seq_len, d = x.shape
    mesh = plsc.VectorSubcoreMesh(core_axis_name="c", subcore_axis_name="s")
    k = functools.partial(
        pl.kernel,
        out_type=jax.ShapeDtypeStruct(x.shape, x.dtype),
        mesh=mesh,
        scratch_types=[
            pltpu.VMEM((_CH, d), jnp.float32),
            pltpu.VMEM((_CH, d), jnp.float32),
        ],
    )(_sc_body)
    return k(x, pe_table[:seq_len])


# speedup vs baseline: 1.2303x; 1.2303x over previous
import functools

import jax
import jax.numpy as jnp
from jax import lax
from jax.experimental import pallas as pl
from jax.experimental.pallas import tpu as pltpu, tpu_sc as plsc

_NC = 2
_NS = 16
_NW = _NC * _NS
_L = 16

_CH = 32
_BS = 2048


def _sc_body(x_hbm, pe_hbm, out_hbm, x_v, pe_v):
    batch, seq_len, d = x_hbm.shape
    rows_w = seq_len // _NW
    nch = rows_w // _CH
    wid = lax.axis_index("s") * _NC + lax.axis_index("c")
    base = wid * rows_w

    def chunk_body(ci, _):
        s0 = base + ci * _CH
        pltpu.sync_copy(pe_hbm.at[pl.ds(s0, _CH)], pe_v)
        pltpu.sync_copy(x_hbm.at[0, pl.ds(s0, _CH)], x_v)

        def row_body(r, _):
            for j in range(d // _L):
                sl = pl.ds(j * _L, _L)
                x_v[r, sl] = x_v[r, sl] + pe_v[r, sl]
            return 0

        lax.fori_loop(0, _CH, row_body, 0)
        pltpu.sync_copy(x_v, out_hbm.at[0, pl.ds(s0, _CH)])
        return 0

    lax.fori_loop(0, nch, chunk_body, 0)


def _tc_body(x_ref, pe_ref, out_ref):
    out_ref[...] = x_ref[...] + pe_ref[...][None]


def kernel(x, pe_table):
    batch, seq_len, d = x.shape
    pe = pe_table[:seq_len]

    mesh = plsc.VectorSubcoreMesh(core_axis_name="c", subcore_axis_name="s")
    sc_k = functools.partial(
        pl.kernel,
        out_type=jax.ShapeDtypeStruct((1, seq_len, d), x.dtype),
        mesh=mesh,
        scratch_types=[
            pltpu.VMEM((_CH, d), jnp.float32),
            pltpu.VMEM((_CH, d), jnp.float32),
        ],
    )(_sc_body)
    sc_out = sc_k(x, pe)

    tc_out = pl.pallas_call(
        _tc_body,
        grid=(seq_len // _BS, batch - 1),
        in_specs=[
            pl.BlockSpec((1, _BS, d), lambda s, b: (b + 1, s, 0)),
            pl.BlockSpec((_BS, d), lambda s, b: (s, 0)),
        ],
        out_specs=pl.BlockSpec((1, _BS, d), lambda s, b: (b, s, 0)),
        out_shape=jax.ShapeDtypeStruct((batch - 1, seq_len, d), x.dtype),
    )(x, pe)

    return jnp.concatenate([sc_out, tc_out], axis=0)


# speedup vs baseline: 1.7607x; 1.4311x over previous
import functools

import jax
import jax.numpy as jnp
from jax import lax
from jax.experimental import pallas as pl
from jax.experimental.pallas import tpu as pltpu, tpu_sc as plsc

_NC = 2
_NS = 16
_NW = _NC * _NS
_L = 16

_CH = 16


def _sc_body(x_hbm, pe_hbm, out_hbm, xb0, xb1, pb0, pb1, ob0, ob1,
             sx0, sx1, sp0, sp1, so0, so1):
    batch, seq_len, d = x_hbm.shape
    rows_w = seq_len // _NW
    nch = rows_w // _CH
    wid = lax.axis_index("s") * _NC + lax.axis_index("c")
    base = wid * rows_w

    xb = (xb0, xb1)
    pb = (pb0, pb1)
    ob = (ob0, ob1)
    sx = (sx0, sx1)
    sp = (sp0, sp1)
    so = (so0, so1)

    def x_in(ci, b, k):
        pltpu.make_async_copy(
            x_hbm.at[b, pl.ds(base + ci * _CH, _CH)], xb[k], sx[k]
        ).start()

    def pe_in(ci, p):
        pltpu.make_async_copy(
            pe_hbm.at[pl.ds(base + ci * _CH, _CH)], pb[p], sp[p]
        ).start()

    def add_chunk(k, p):
        def row_body(r, _):
            for j in range(d // _L):
                sl = pl.ds(j * _L, _L)
                ob[k][r, sl] = xb[k][r, sl] + pb[p][r, sl]
            return 0

        lax.fori_loop(0, _CH, row_body, 0)

    pe_in(0, 0)
    x_in(0, 0, 0)
    x_in(0, 1, 1)

    def chunk_pair(i, _):
        for cpar in (0, 1):
            ci = 2 * i + cpar
            @pl.when(ci + 1 < nch)
            def _():
                pe_in(ci + 1, 1 - cpar)

            pltpu.make_async_copy(
                pe_hbm.at[pl.ds(base, _CH)], pb[cpar], sp[cpar]
            ).wait()

            for b in range(batch):
                k = b % 2
                pltpu.make_async_copy(
                    x_hbm.at[b, pl.ds(base, _CH)], xb[k], sx[k]
                ).wait()
                if b >= 2:
                    pltpu.make_async_copy(
                        ob[k], out_hbm.at[b, pl.ds(base, _CH)], so[k]
                    ).wait()
                else:
                    @pl.when(ci > 0)
                    def _():
                        pltpu.make_async_copy(
                            ob[k], out_hbm.at[b, pl.ds(base, _CH)], so[k]
                        ).wait()

                add_chunk(k, cpar)

                if b < 2:
                    x_in(ci, b + 2, k)
                else:
                    @pl.when(ci + 1 < nch)
                    def _():
                        x_in(ci + 1, b - 2, k)

                pltpu.make_async_copy(
                    ob[k], out_hbm.at[b, pl.ds(base + ci * _CH, _CH)], so[k]
                ).start()
        return 0

    lax.fori_loop(0, nch // 2, chunk_pair, 0)

    for k in range(2):
        pltpu.make_async_copy(
            ob[k], out_hbm.at[0, pl.ds(base, _CH)], so[k]
        ).wait()


def kernel(x, pe_table):
    batch, seq_len, d = x.shape
    mesh = plsc.VectorSubcoreMesh(core_axis_name="c", subcore_axis_name="s")
    k = functools.partial(
        pl.kernel,
        out_type=jax.ShapeDtypeStruct(x.shape, x.dtype),
        mesh=mesh,
        scratch_types=[
            pltpu.VMEM((_CH, d), jnp.float32),
            pltpu.VMEM((_CH, d), jnp.float32),
            pltpu.VMEM((_CH, d), jnp.float32),
            pltpu.VMEM((_CH, d), jnp.float32),
            pltpu.VMEM((_CH, d), jnp.float32),
            pltpu.VMEM((_CH, d), jnp.float32),
            pltpu.SemaphoreType.DMA,
            pltpu.SemaphoreType.DMA,
            pltpu.SemaphoreType.DMA,
            pltpu.SemaphoreType.DMA,
            pltpu.SemaphoreType.DMA,
            pltpu.SemaphoreType.DMA,
        ],
    )(_sc_body)
    return k(x, pe_table[:seq_len])
